# Initial kernel scaffold; baseline (speedup 1.0000x reference)
import jax
import jax.numpy as jnp
from jax.experimental import pallas as pl


def kernel(x, edge_index, pos, batch, W1, b1, W2, b2, linW, linb, lin2W, lin2b):
    raise NotImplementedError("write your pallas kernel here")


# speedup vs baseline: 12.3079x; 12.3079x over previous
import functools
import jax
import jax.numpy as jnp
from jax import lax
from jax.experimental import pallas as pl
from jax.experimental.pallas import tpu as pltpu
from jax.experimental.pallas import tpu_sc as plsc

N = 10000
E = 320000
D = 128
CLS = 64
G = 64
PAD_N = 10240
NC = 2
NS = 16
NW = NC * NS
EPT = E // NW
EBLK = 40
NCHUNK = 5
NEB = EPT // EBLK
ROWS_PS = PAD_N // NS
ROWS_PT = PAD_N // NW
PCHUNK = 64

_mesh = plsc.VectorSubcoreMesh(core_axis_name="c", subcore_axis_name="s")
f32 = jnp.float32
i32 = jnp.int32


def _wid():
    return lax.axis_index("s") * NC + lax.axis_index("c")


def _deg_body(dst_hbm, zeros16_hbm, ones_hbm, deg_out, ones_v, didx, deg_sh, sem):
    cid = lax.axis_index("c")
    sid = lax.axis_index("s")
    wid = sid * NC + cid
    e0 = wid * EPT
    pltpu.sync_copy(zeros16_hbm, deg_sh.at[pl.ds(sid * ROWS_PS, ROWS_PS)])
    pltpu.sync_copy(ones_hbm, ones_v)
    plsc.subcore_barrier()

    def blk(b, carry):
        pltpu.sync_copy(dst_hbm.at[pl.ds(e0 + b * EBLK, EBLK)], didx)
        pltpu.sync_copy(ones_v, deg_sh.at[didx], add=True)
        return carry

    lax.fori_loop(0, NEB, blk, 0)
    plsc.subcore_barrier()
    rows = pl.ds(sid * ROWS_PS, ROWS_PS)
    pltpu.sync_copy(deg_sh.at[rows], deg_out.at[cid, rows])


_deg_call = pl.kernel(
    _deg_body,
    out_type=jax.ShapeDtypeStruct((NC, PAD_N, 16), f32),
    mesh=_mesh,
    scratch_types=[
        pltpu.VMEM((EBLK, 16), f32),
        pltpu.VMEM((EBLK,), i32),
        pltpu.VMEM_SHARED((PAD_N, 16), f32),
        pltpu.SemaphoreType.DMA,
    ],
)


def _scat_body(y_hbm, src_hbm, dst_hbm, zeros_hbm, acc_out, sidx, didx, rows_v, acc_sh, gsems):
    cid = lax.axis_index("c")
    sid = lax.axis_index("s")
    wid = sid * NC + cid
    e0 = wid * EPT
    pltpu.sync_copy(zeros_hbm, acc_sh.at[pl.ds(sid * ROWS_PS, ROWS_PS)])
    plsc.subcore_barrier()

    def chunk(c, carry):
        descs = []
        for j in range(NCHUNK):
            b = c * NCHUNK + j
            pltpu.sync_copy(src_hbm.at[pl.ds(e0 + b * EBLK, EBLK)], sidx[j])
            pltpu.sync_copy(dst_hbm.at[pl.ds(e0 + b * EBLK, EBLK)], didx[j])
            descs.append(pltpu.async_copy(y_hbm.at[sidx[j]], rows_v[j], gsems[j]))
        for j in range(NCHUNK):
            descs[j].wait()
            pltpu.sync_copy(rows_v[j], acc_sh.at[didx[j]], add=True)
        return carry

    lax.fori_loop(0, NEB // NCHUNK, chunk, 0)
    plsc.subcore_barrier()
    rows = pl.ds(sid * ROWS_PS, ROWS_PS)
    pltpu.sync_copy(acc_sh.at[rows], acc_out.at[cid, rows])


_scat_call = pl.kernel(
    _scat_body,
    out_type=jax.ShapeDtypeStruct((NC, PAD_N, D), f32),
    mesh=_mesh,
    scratch_types=[
        [pltpu.VMEM((EBLK,), i32) for _ in range(NCHUNK)],
        [pltpu.VMEM((EBLK,), i32) for _ in range(NCHUNK)],
        [pltpu.VMEM((EBLK, D), f32) for _ in range(NCHUNK)],
        pltpu.VMEM_SHARED((PAD_N, D), f32),
        [pltpu.SemaphoreType.DMA for _ in range(NCHUNK)],
    ],
)


def _l1_body(x_ref, w_ref, deg_ref, y_ref, dinv_ref):
    deg = deg_ref[0, :, :1] + deg_ref[1, :, :1]
    dinv = lax.rsqrt(deg + 1.0)
    xw = jnp.dot(x_ref[...], w_ref[...], preferred_element_type=f32)
    y_ref[...] = dinv * xw
    dinv_ref[...] = jnp.broadcast_to(dinv, dinv_ref.shape)


def _l1_call(xp, W1, degAB):
    blk = PAD_N // 8
    return pl.pallas_call(
        _l1_body,
        grid=(8,),
        in_specs=[
            pl.BlockSpec((blk, D), lambda i: (i, 0)),
            pl.BlockSpec((D, D), lambda i: (0, 0)),
            pl.BlockSpec((NC, blk, 16), lambda i: (0, i, 0)),
        ],
        out_specs=[
            pl.BlockSpec((blk, D), lambda i: (i, 0)),
            pl.BlockSpec((blk, 16), lambda i: (i, 0)),
        ],
        out_shape=[
            jax.ShapeDtypeStruct((PAD_N, D), f32),
            jax.ShapeDtypeStruct((PAD_N, 16), f32),
        ],
    )(xp, W1, degAB)


def _l2_body(acc_ref, y1_ref, dinv_ref, b1_ref, w2_ref, y2_ref):
    dinv = dinv_ref[:, :1]
    h1 = jnp.maximum(dinv * (acc_ref[0] + acc_ref[1] + y1_ref[...]) + b1_ref[...], 0.0)
    y2_ref[...] = dinv * jnp.dot(h1, w2_ref[...], preferred_element_type=f32)


def _l2_call(accAB, y1, dinvb, b1r, W2):
    blk = PAD_N // 8
    return pl.pallas_call(
        _l2_body,
        grid=(8,),
        in_specs=[
            pl.BlockSpec((NC, blk, D), lambda i: (0, i, 0)),
            pl.BlockSpec((blk, D), lambda i: (i, 0)),
            pl.BlockSpec((blk, 16), lambda i: (i, 0)),
            pl.BlockSpec((1, D), lambda i: (0, 0)),
            pl.BlockSpec((D, D), lambda i: (0, 0)),
        ],
        out_specs=pl.BlockSpec((blk, D), lambda i: (i, 0)),
        out_shape=jax.ShapeDtypeStruct((PAD_N, D), f32),
    )(accAB, y1, dinvb, b1r, W2)


def _pool_body(acc_hbm, y2_hbm, dinv_hbm, batch_hbm, b2_hbm,
               psum, pmax, pcnt,
               a0c, a1c, y2c, bat, dnv, b2v, lsum, lmax, lcnt):
    wid = _wid()
    n0 = wid * ROWS_PT
    valid = jnp.clip(N - n0, 0, ROWS_PT)
    pltpu.sync_copy(batch_hbm.at[pl.ds(n0 * 16, ROWS_PT * 16)], bat)
    pltpu.sync_copy(dinv_hbm.at[pl.ds(n0 * 16, ROWS_PT * 16)], dnv)
    pltpu.sync_copy(b2_hbm, b2v)

    def zrow(i, carry):
        z = jnp.zeros((16,), f32)
        lsum[pl.ds(i * 16, 16)] = z
        lmax[pl.ds(i * 16, 16)] = z
        return carry

    lax.fori_loop(0, G * D // 16, zrow, 0)

    def zc(i, carry):
        lcnt[pl.ds(i * 16, 16)] = jnp.zeros((16,), f32)
        return carry

    lax.fori_loop(0, G, zc, 0)

    for c in range(ROWS_PT // PCHUNK):
        r0 = n0 + c * PCHUNK
        pltpu.sync_copy(acc_hbm.at[pl.ds(r0 * D, PCHUNK * D)], a0c)
        pltpu.sync_copy(acc_hbm.at[pl.ds(PAD_N * D + r0 * D, PCHUNK * D)], a1c)
        pltpu.sync_copy(y2_hbm.at[pl.ds(r0 * D, PCHUNK * D)], y2c)
        nloc = jnp.clip(valid - c * PCHUNK, 0, PCHUNK)

        def node(j, carry, c=c):
            jg = c * PCHUNK + j
            g = bat[pl.ds(jg * 16, 16)][0]
            dv = dnv[pl.ds(jg * 16, 16)]
            base = g * D
            for f in range(D // 16):
                off = j * D + f * 16
                h = dv * (a0c[pl.ds(off, 16)] + a1c[pl.ds(off, 16)] + y2c[pl.ds(off, 16)])
                h = jnp.maximum(h + b2v[pl.ds(f * 16, 16)], 0.0)
                sl = pl.ds(base + f * 16, 16)
                lsum[sl] = lsum[sl] + h
                lmax[sl] = jnp.maximum(lmax[sl], h)
            cs = pl.ds(g * 16, 16)
            lcnt[cs] = lcnt[cs] + 1.0
            return carry

        lax.fori_loop(0, nloc, node, 0)

    pltpu.sync_copy(lsum, psum.at[wid])
    pltpu.sync_copy(lmax, pmax.at[wid])
    pltpu.sync_copy(lcnt, pcnt.at[wid])


_pool_call = pl.kernel(
    _pool_body,
    out_type=(
        jax.ShapeDtypeStruct((NW, G * D), f32),
        jax.ShapeDtypeStruct((NW, G * D), f32),
        jax.ShapeDtypeStruct((NW, G * 16), f32),
    ),
    mesh=_mesh,
    scratch_types=[
        pltpu.VMEM((PCHUNK * D,), f32),
        pltpu.VMEM((PCHUNK * D,), f32),
        pltpu.VMEM((PCHUNK * D,), f32),
        pltpu.VMEM((ROWS_PT * 16,), i32),
        pltpu.VMEM((ROWS_PT * 16,), f32),
        pltpu.VMEM((D,), f32),
        pltpu.VMEM((G * D,), f32),
        pltpu.VMEM((G * D,), f32),
        pltpu.VMEM((G * 16,), f32),
    ],
)


def _head_body(psum_ref, pmax_ref, pcnt_ref, linW_ref, linb_ref, lin2W_ref, lin2b_ref, out_ref):
    ps = psum_ref[...].reshape(NW, G, D)
    pm = pmax_ref[...].reshape(NW, G, D)
    seg_sum = ps.sum(axis=0)
    seg_max = pm.max(axis=0)
    cnt = pcnt_ref[...].reshape(NW, G, 16).sum(axis=0)[:, :1]
    mean = seg_sum / jnp.maximum(cnt, 1.0)
    gfeat = jnp.concatenate([mean, seg_max, seg_sum], axis=1)
    z = jnp.maximum(jnp.dot(gfeat, linW_ref[...], preferred_element_type=f32) + linb_ref[...], 0.0)
    out_ref[...] = jax.nn.sigmoid(jnp.dot(z, lin2W_ref[...], preferred_element_type=f32) + lin2b_ref[...])


def _head_call(psum, pmax, pcnt, linW, linbr, lin2Wp, lin2bp):
    return pl.pallas_call(
        _head_body,
        out_shape=jax.ShapeDtypeStruct((G, 128), f32),
    )(psum, pmax, pcnt, linW, linbr, lin2Wp, lin2bp)


@jax.jit
def kernel(x, edge_index, pos, batch, W1, b1, W2, b2, linW, linb, lin2W, lin2b):
    del pos
    ei = edge_index.astype(i32)
    src = ei[0]
    dst = ei[1]
    xp = jnp.zeros((PAD_N, D), f32).at[:N].set(x)
    batchb = jnp.zeros((PAD_N, 16), i32).at[:N].set(
        jnp.broadcast_to(batch.astype(i32)[:, None], (N, 16)))
    zeros640 = jnp.zeros((ROWS_PS, D), f32)
    zeros16 = jnp.zeros((ROWS_PS, 16), f32)
    ones80 = jnp.ones((EBLK, 16), f32)

    degAB = _deg_call(dst, zeros16, ones80)
    y1, dinvb = _l1_call(xp, W1, degAB)
    accAB = _scat_call(y1, src, dst, zeros640)
    y2 = _l2_call(accAB, y1, dinvb, b1.reshape(1, D), W2)
    accAB2 = _scat_call(y2, src, dst, zeros640)
    psum, pmax, pcnt = _pool_call(
        accAB2.reshape(-1), y2.reshape(-1),
        dinvb.reshape(-1), batchb.reshape(-1), b2)
    lin2Wp = jnp.concatenate([lin2W, jnp.zeros((CLS, 127), f32)], axis=1)
    lin2bp = jnp.broadcast_to(lin2b.reshape(1, 1), (1, 128))
    out = _head_call(psum, pmax, pcnt, linW, linb.reshape(1, CLS), lin2Wp, lin2bp)
    return out[:, :1]
